# baseline (speedup 1.0000x reference)
import functools

import jax
import jax.numpy as jnp
from jax import lax
from jax.experimental import pallas as pl
from jax.experimental.pallas import tpu as pltpu
from jax.experimental.pallas import tpu_sc as plsc

_N = 50000
_B = 4
_VT_MAX = 0.385
_DT = 0.015625

_NW = 32
_SUB = 128
_JJ = 13
_C = _SUB * _JJ
_NPAD = _NW * _C
_ROWS = _NPAD // 128
_TB = 32
_KITERS = _C // 16
_SENT = 1e30

_f32 = jnp.float32
_i32 = jnp.int32


def _pad_flat(x, fill=0):
    x = x.reshape(-1)
    return jnp.pad(x, (0, _NPAD - x.shape[0]), constant_values=fill)


def _pad2d(x, fill=0):
    return _pad_flat(x, fill).reshape(_ROWS, 128)


def _prep_body(relax_ref, py, nt, uold, uori, jo, bat, xe, ye,
               g0, g1, g2, g3, gj, nte,
               d_o, uvis_o, ubase_o, urel_o, u_o, v_o, mextf_o):
    ubc = -jnp.tanh(py[...] * 0.5)
    mask_b = (nt[...] == 1) | (nt[...] == 2)
    d_o[...] = jnp.where(mask_b, ubc, jnp.float32(jnp.inf))
    uvis_o[...] = jnp.where(mask_b, ubc, uori[...])
    ubase_o[...] = (uori[...] - uold[...]) * (1.0 / _DT) / jo[...]
    b = bat[...]
    r0 = relax_ref[0, 0]
    r1 = relax_ref[0, 1]
    r2 = relax_ref[0, 2]
    r3 = relax_ref[0, 3]
    urel_o[...] = jnp.where(b == 0, r0,
                  jnp.where(b == 1, r1,
                  jnp.where(b == 2, r2, r3))).astype(_f32)
    x = xe[...]
    y = ye[...]
    r = jnp.sqrt(x * x + y * y)
    er = jnp.exp(-r)
    e2 = er * er
    sech = (2.0 * er) / (1.0 + e2)
    v_t = sech * sech * ((1.0 - e2) / (1.0 + e2))
    mask_r = r > 1e-12
    r_safe = jnp.where(mask_r, r, 1.0)
    omega = jnp.where(mask_r, (v_t / r_safe) / _VT_MAX, 0.0)
    a = -omega * y
    bb = omega * x
    u_o[...] = (a * g0[...] + bb * g1[...]) / gj[...]
    v_o[...] = (a * g2[...] + bb * g3[...]) / gj[...]
    mextf_o[...] = jnp.where((nte[...] == 1) | (nte[...] == 2), 1.0, 0.0)


def _tc_prep(relax, py, nt, uold, uori, jo, bat, xe, ye, g0, g1, g2, g3, gj,
             nte):
    blk = lambda: pl.BlockSpec((_TB, 128), lambda i: (i, 0))
    n_in = 14
    n_out = 7
    return pl.pallas_call(
        _prep_body,
        grid=(_ROWS // _TB,),
        in_specs=[pl.BlockSpec(memory_space=pltpu.SMEM)]
        + [blk() for _ in range(n_in)],
        out_specs=[blk() for _ in range(n_out)],
        out_shape=[jax.ShapeDtypeStruct((_ROWS, 128), _f32)
                   for _ in range(n_out)],
    )(relax, py, nt, uold, uori, jo, bat, xe, ye, g0, g1, g2, g3, gj, nte)


@functools.lru_cache(maxsize=None)
def _mesh():
    return plsc.VectorSubcoreMesh(core_axis_name="c", subcore_axis_name="s",
                                  num_cores=2, num_subcores=16)


def _wid():
    return lax.axis_index("s") * 2 + lax.axis_index("c")


def _fire(tbl, idx_v, dst_v, sem):
    return [pltpu.async_copy(tbl.at[idx_v.at[pl.ds(j * _SUB, _SUB)]],
                             dst_v.at[pl.ds(j * _SUB, _SUB)], sem)
            for j in range(_JJ)]


def _p1_body(uold_t, uori_t, d_t, eidx, mextf,
             uo_out, un_out,
             idx_v, a_v, b_v, d_v, me_v, uo_v, un_v, sem):
    w = _wid()
    base = w * _C
    pltpu.sync_copy(eidx.at[pl.ds(base, _C)], idx_v)
    descs = _fire(uold_t, idx_v, a_v, sem)
    descs += _fire(uori_t, idx_v, b_v, sem)
    pltpu.sync_copy(mextf.at[pl.ds(base, _C)], me_v)
    for d in descs:
        d.wait()
    descs = _fire(d_t, idx_v, d_v, sem)
    for d in descs:
        d.wait()

    def body(k, carry):
        s = pl.ds(k * 16, 16)
        dg = d_v[s]
        sel = (me_v[s] > 0.5) & (dg < _SENT)
        uo_v[s] = jnp.where(sel, dg, a_v[s])
        un_v[s] = jnp.where(sel, dg, b_v[s])
        return carry

    lax.fori_loop(0, _KITERS, body, 0)
    pltpu.sync_copy(uo_v, uo_out.at[pl.ds(base, _C)])
    pltpu.sync_copy(un_v, un_out.at[pl.ds(base, _C)])


@functools.lru_cache(maxsize=None)
def _p1():
    return pl.kernel(
        _p1_body,
        out_type=[jax.ShapeDtypeStruct((_NPAD,), _f32),
                  jax.ShapeDtypeStruct((_NPAD,), _f32)],
        mesh=_mesh(),
        scratch_types=[
            pltpu.VMEM((_C,), _i32),
            pltpu.VMEM((_C,), _f32),
            pltpu.VMEM((_C,), _f32),
            pltpu.VMEM((_C,), _f32),
            pltpu.VMEM((_C,), _f32),
            pltpu.VMEM((_C,), _f32),
            pltpu.VMEM((_C,), _f32),
            pltpu.SemaphoreType.DMA,
        ],
    )


def _p2_body(uo_t, un_t, uu_t, vv_t, exl, exr, eed, eeu,
             fxo_out, fxn_out, feo_out, fen_out,
             il_v, ir_v, uol_v, uor_v, unl_v, unr_v, cl_v, cr_v,
             fo_v, fn_v, sem):
    w = _wid()
    base = w * _C
    for il, ir, ct, oo, on in ((exl, exr, uu_t, fxo_out, fxn_out),
                               (eed, eeu, vv_t, feo_out, fen_out)):
        pltpu.sync_copy(il.at[pl.ds(base, _C)], il_v)
        pltpu.sync_copy(ir.at[pl.ds(base, _C)], ir_v)
        descs = _fire(uo_t, il_v, uol_v, sem)
        descs += _fire(uo_t, ir_v, uor_v, sem)
        for d in descs:
            d.wait()
        descs = _fire(un_t, il_v, unl_v, sem)
        descs += _fire(un_t, ir_v, unr_v, sem)
        for d in descs:
            d.wait()
        descs = _fire(ct, il_v, cl_v, sem)
        descs += _fire(ct, ir_v, cr_v, sem)
        for d in descs:
            d.wait()

        def body(k, carry):
            s = pl.ds(k * 16, 16)
            cc = 0.25 * (cl_v[s] + cr_v[s])
            fo_v[s] = (uol_v[s] + uor_v[s]) * cc
            fn_v[s] = (unl_v[s] + unr_v[s]) * cc
            return carry

        lax.fori_loop(0, _KITERS, body, 0)
        pltpu.sync_copy(fo_v, oo.at[pl.ds(base, _C)])
        pltpu.sync_copy(fn_v, on.at[pl.ds(base, _C)])


@functools.lru_cache(maxsize=None)
def _p2():
    return pl.kernel(
        _p2_body,
        out_type=[jax.ShapeDtypeStruct((_NPAD,), _f32) for _ in range(4)],
        mesh=_mesh(),
        scratch_types=[
            pltpu.VMEM((_C,), _i32),
            pltpu.VMEM((_C,), _i32),
        ] + [pltpu.VMEM((_C,), _f32) for _ in range(8)] + [
            pltpu.SemaphoreType.DMA,
        ],
    )


def _p3_body(fxo_t, fxn_t, feo_t, fen_t, fxl, fxr, fed, feu, ubase, urel,
             loss_out,
             ia, ib, ic, idd, xol, xor_, xnl, xnr, eod, eou, end_, enu,
             ub_v, ur_v, loss_v, sem):
    w = _wid()
    base = w * _C
    pltpu.sync_copy(fxl.at[pl.ds(base, _C)], ia)
    pltpu.sync_copy(fxr.at[pl.ds(base, _C)], ib)
    pltpu.sync_copy(fed.at[pl.ds(base, _C)], ic)
    pltpu.sync_copy(feu.at[pl.ds(base, _C)], idd)
    descs = _fire(fxo_t, ia, xol, sem)
    descs += _fire(fxo_t, ib, xor_, sem)
    for d in descs:
        d.wait()
    descs = _fire(fxn_t, ia, xnl, sem)
    descs += _fire(fxn_t, ib, xnr, sem)
    for d in descs:
        d.wait()
    descs = _fire(feo_t, ic, eod, sem)
    descs += _fire(feo_t, idd, eou, sem)
    for d in descs:
        d.wait()
    descs = _fire(fen_t, ic, end_, sem)
    descs += _fire(fen_t, idd, enu, sem)
    pltpu.sync_copy(ubase.at[pl.ds(base, _C)], ub_v)
    pltpu.sync_copy(urel.at[pl.ds(base, _C)], ur_v)
    for d in descs:
        d.wait()

    def body(k, carry):
        s = pl.ds(k * 16, 16)
        co = (xor_[s] - xol[s]) + (eou[s] - eod[s])
        cn = (xnr[s] - xnl[s]) + (enu[s] - end_[s])
        ur = ur_v[s]
        loss_v[s] = ub_v[s] + ur * co + (1.0 - ur) * cn
        return carry

    lax.fori_loop(0, _KITERS, body, 0)
    pltpu.sync_copy(loss_v, loss_out.at[pl.ds(base, _C)])


@functools.lru_cache(maxsize=None)
def _p3():
    return pl.kernel(
        _p3_body,
        out_type=jax.ShapeDtypeStruct((_NPAD,), _f32),
        mesh=_mesh(),
        scratch_types=[
            pltpu.VMEM((_C,), _i32) for _ in range(4)
        ] + [pltpu.VMEM((_C,), _f32) for _ in range(11)] + [
            pltpu.SemaphoreType.DMA,
        ],
    )


def kernel(original_u, u_old, pos, node_type, extend_index, node_type_extended,
           original_block_metrics, extended_block_metrics, pos_extended,
           edge_node_xi_index, edge_node_eta_index, face_xi, face_eta, batch,
           pde_theta, relaxtion):
    del pde_theta
    py = _pad2d(pos[:, 1].astype(_f32))
    nt = _pad2d(node_type.astype(_i32))
    uold_f = _pad_flat(u_old[:, 0].astype(_f32))
    uori_f = _pad_flat(original_u[:, 0].astype(_f32))
    jo = _pad2d(original_block_metrics[:, 4].astype(_f32), fill=1)
    bat = _pad2d(batch.astype(_i32))
    xe = _pad2d(pos_extended[:, 0].astype(_f32))
    ye = _pad2d(pos_extended[:, 1].astype(_f32))
    g0 = _pad2d(extended_block_metrics[:, 0].astype(_f32))
    g1 = _pad2d(extended_block_metrics[:, 1].astype(_f32))
    g2 = _pad2d(extended_block_metrics[:, 2].astype(_f32))
    g3 = _pad2d(extended_block_metrics[:, 3].astype(_f32))
    gj = _pad2d(extended_block_metrics[:, 4].astype(_f32), fill=1)
    nte = _pad2d(node_type_extended.astype(_i32))
    relax = relaxtion.astype(_f32).reshape(1, _B)

    d_t, uvis, ubase, urel, uu, vv, mextf = _tc_prep(
        relax, py, nt, uold_f.reshape(_ROWS, 128), uori_f.reshape(_ROWS, 128),
        jo, bat, xe, ye, g0, g1, g2, g3, gj, nte)

    eidx = _pad_flat(extend_index.astype(_i32))
    uo_t, un_t = _p1()(uold_f, uori_f, d_t.reshape(-1), eidx,
                       mextf.reshape(-1))

    exl = _pad_flat(edge_node_xi_index[0].astype(_i32))
    exr = _pad_flat(edge_node_xi_index[1].astype(_i32))
    eed = _pad_flat(edge_node_eta_index[0].astype(_i32))
    eeu = _pad_flat(edge_node_eta_index[1].astype(_i32))
    fxo_t, fxn_t, feo_t, fen_t = _p2()(uo_t, un_t, uu.reshape(-1),
                                       vv.reshape(-1), exl, exr, eed, eeu)

    fxl = _pad_flat(face_xi[0].astype(_i32))
    fxr = _pad_flat(face_xi[1].astype(_i32))
    fed = _pad_flat(face_eta[0].astype(_i32))
    feu = _pad_flat(face_eta[1].astype(_i32))
    loss = _p3()(fxo_t, fxn_t, feo_t, fen_t, fxl, fxr, fed, feu,
                 ubase.reshape(-1), urel.reshape(-1))

    return (loss[:_N].reshape(_N, 1), uvis.reshape(-1)[:_N].reshape(_N, 1))


# speedup vs baseline: 1.0499x; 1.0499x over previous
import functools

import jax
import jax.numpy as jnp
from jax import lax
from jax.experimental import pallas as pl
from jax.experimental.pallas import tpu as pltpu
from jax.experimental.pallas import tpu_sc as plsc

_N = 50000
_B = 4
_VT_MAX = 0.385
_DT = 0.015625

_NW = 32
_SUB = 128
_JJ = 13
_C = _SUB * _JJ
_NPAD = _NW * _C
_ROWS = _NPAD // 128
_TB = 32
_KITERS = _C // 16
_SENT = 1e30

_f32 = jnp.float32
_i32 = jnp.int32


def _pad_flat(x, fill=0):
    x = x.reshape(-1)
    return jnp.pad(x, (0, _NPAD - x.shape[0]), constant_values=fill)


def _pad2d(x, fill=0):
    return _pad_flat(x, fill).reshape(_ROWS, 128)


def _prep_body(relax_ref, py, nt, uold, uori, jo, bat, xe, ye,
               g0, g1, g2, g3, gj, nte,
               d_o, uvis_o, ubase_o, urel_o, u_o, v_o, mextf_o):
    ubc = -jnp.tanh(py[...] * 0.5)
    mask_b = (nt[...] == 1) | (nt[...] == 2)
    d_o[...] = jnp.where(mask_b, ubc, jnp.float32(jnp.inf))
    uvis_o[...] = jnp.where(mask_b, ubc, uori[...])
    ubase_o[...] = (uori[...] - uold[...]) * (1.0 / _DT) / jo[...]
    b = bat[...]
    r0 = relax_ref[0, 0]
    r1 = relax_ref[0, 1]
    r2 = relax_ref[0, 2]
    r3 = relax_ref[0, 3]
    urel_o[...] = jnp.where(b == 0, r0,
                  jnp.where(b == 1, r1,
                  jnp.where(b == 2, r2, r3))).astype(_f32)
    x = xe[...]
    y = ye[...]
    r = jnp.sqrt(x * x + y * y)
    er = jnp.exp(-r)
    e2 = er * er
    sech = (2.0 * er) / (1.0 + e2)
    v_t = sech * sech * ((1.0 - e2) / (1.0 + e2))
    mask_r = r > 1e-12
    r_safe = jnp.where(mask_r, r, 1.0)
    omega = jnp.where(mask_r, (v_t / r_safe) / _VT_MAX, 0.0)
    a = -omega * y
    bb = omega * x
    u_o[...] = (a * g0[...] + bb * g1[...]) / gj[...]
    v_o[...] = (a * g2[...] + bb * g3[...]) / gj[...]
    mextf_o[...] = jnp.where((nte[...] == 1) | (nte[...] == 2), 1.0, 0.0)


def _tc_prep(relax, py, nt, uold, uori, jo, bat, xe, ye, g0, g1, g2, g3, gj,
             nte):
    blk = lambda: pl.BlockSpec((_TB, 128), lambda i: (i, 0))
    n_in = 14
    n_out = 7
    return pl.pallas_call(
        _prep_body,
        grid=(_ROWS // _TB,),
        in_specs=[pl.BlockSpec(memory_space=pltpu.SMEM)]
        + [blk() for _ in range(n_in)],
        out_specs=[blk() for _ in range(n_out)],
        out_shape=[jax.ShapeDtypeStruct((_ROWS, 128), _f32)
                   for _ in range(n_out)],
    )(relax, py, nt, uold, uori, jo, bat, xe, ye, g0, g1, g2, g3, gj, nte)


@functools.lru_cache(maxsize=None)
def _mesh():
    return plsc.VectorSubcoreMesh(core_axis_name="c", subcore_axis_name="s",
                                  num_cores=2, num_subcores=16)


def _wid():
    return lax.axis_index("s") * 2 + lax.axis_index("c")


def _fire(tbl, idx_v, dst_v, sem):
    return [pltpu.async_copy(tbl.at[idx_v], dst_v, sem)]


def _p1_body(uold_t, uori_t, d_t, eidx, mextf,
             uo_out, un_out,
             idx_v, a_v, b_v, d_v, me_v, uo_v, un_v, sem):
    w = _wid()
    base = w * _C
    pltpu.sync_copy(eidx.at[pl.ds(base, _C)], idx_v)
    descs = _fire(uold_t, idx_v, a_v, sem)
    descs += _fire(uori_t, idx_v, b_v, sem)
    descs += _fire(d_t, idx_v, d_v, sem)
    pltpu.sync_copy(mextf.at[pl.ds(base, _C)], me_v)
    for d in descs:
        d.wait()

    def body(k, carry):
        s = pl.ds(k * 16, 16)
        dg = d_v[s]
        sel = (me_v[s] > 0.5) & (dg < _SENT)
        uo_v[s] = jnp.where(sel, dg, a_v[s])
        un_v[s] = jnp.where(sel, dg, b_v[s])
        return carry

    lax.fori_loop(0, _KITERS, body, 0)
    pltpu.sync_copy(uo_v, uo_out.at[pl.ds(base, _C)])
    pltpu.sync_copy(un_v, un_out.at[pl.ds(base, _C)])


@functools.lru_cache(maxsize=None)
def _p1():
    return pl.kernel(
        _p1_body,
        out_type=[jax.ShapeDtypeStruct((_NPAD,), _f32),
                  jax.ShapeDtypeStruct((_NPAD,), _f32)],
        mesh=_mesh(),
        scratch_types=[
            pltpu.VMEM((_C,), _i32),
            pltpu.VMEM((_C,), _f32),
            pltpu.VMEM((_C,), _f32),
            pltpu.VMEM((_C,), _f32),
            pltpu.VMEM((_C,), _f32),
            pltpu.VMEM((_C,), _f32),
            pltpu.VMEM((_C,), _f32),
            pltpu.SemaphoreType.DMA,
        ],
    )


def _p2_body(uo_t, un_t, uu_t, vv_t, exl, exr, eed, eeu,
             fxo_out, fxn_out, feo_out, fen_out,
             il_v, ir_v, uol_v, uor_v, unl_v, unr_v, cl_v, cr_v,
             fo_v, fn_v, sem):
    w = _wid()
    base = w * _C
    for il, ir, ct, oo, on in ((exl, exr, uu_t, fxo_out, fxn_out),
                               (eed, eeu, vv_t, feo_out, fen_out)):
        pltpu.sync_copy(il.at[pl.ds(base, _C)], il_v)
        pltpu.sync_copy(ir.at[pl.ds(base, _C)], ir_v)
        descs = _fire(uo_t, il_v, uol_v, sem)
        descs += _fire(uo_t, ir_v, uor_v, sem)
        descs += _fire(un_t, il_v, unl_v, sem)
        descs += _fire(un_t, ir_v, unr_v, sem)
        descs += _fire(ct, il_v, cl_v, sem)
        descs += _fire(ct, ir_v, cr_v, sem)
        for d in descs:
            d.wait()

        def body(k, carry):
            s = pl.ds(k * 16, 16)
            cc = 0.25 * (cl_v[s] + cr_v[s])
            fo_v[s] = (uol_v[s] + uor_v[s]) * cc
            fn_v[s] = (unl_v[s] + unr_v[s]) * cc
            return carry

        lax.fori_loop(0, _KITERS, body, 0)
        pltpu.sync_copy(fo_v, oo.at[pl.ds(base, _C)])
        pltpu.sync_copy(fn_v, on.at[pl.ds(base, _C)])


@functools.lru_cache(maxsize=None)
def _p2():
    return pl.kernel(
        _p2_body,
        out_type=[jax.ShapeDtypeStruct((_NPAD,), _f32) for _ in range(4)],
        mesh=_mesh(),
        scratch_types=[
            pltpu.VMEM((_C,), _i32),
            pltpu.VMEM((_C,), _i32),
        ] + [pltpu.VMEM((_C,), _f32) for _ in range(8)] + [
            pltpu.SemaphoreType.DMA,
        ],
    )


def _p3_body(fxo_t, fxn_t, feo_t, fen_t, fxl, fxr, fed, feu, ubase, urel,
             loss_out,
             ia, ib, ic, idd, xol, xor_, xnl, xnr, eod, eou, end_, enu,
             ub_v, ur_v, loss_v, sem):
    w = _wid()
    base = w * _C
    pltpu.sync_copy(fxl.at[pl.ds(base, _C)], ia)
    pltpu.sync_copy(fxr.at[pl.ds(base, _C)], ib)
    pltpu.sync_copy(fed.at[pl.ds(base, _C)], ic)
    pltpu.sync_copy(feu.at[pl.ds(base, _C)], idd)
    descs = _fire(fxo_t, ia, xol, sem)
    descs += _fire(fxo_t, ib, xor_, sem)
    descs += _fire(fxn_t, ia, xnl, sem)
    descs += _fire(fxn_t, ib, xnr, sem)
    descs += _fire(feo_t, ic, eod, sem)
    descs += _fire(feo_t, idd, eou, sem)
    descs += _fire(fen_t, ic, end_, sem)
    descs += _fire(fen_t, idd, enu, sem)
    pltpu.sync_copy(ubase.at[pl.ds(base, _C)], ub_v)
    pltpu.sync_copy(urel.at[pl.ds(base, _C)], ur_v)
    for d in descs:
        d.wait()

    def body(k, carry):
        s = pl.ds(k * 16, 16)
        co = (xor_[s] - xol[s]) + (eou[s] - eod[s])
        cn = (xnr[s] - xnl[s]) + (enu[s] - end_[s])
        ur = ur_v[s]
        loss_v[s] = ub_v[s] + ur * co + (1.0 - ur) * cn
        return carry

    lax.fori_loop(0, _KITERS, body, 0)
    pltpu.sync_copy(loss_v, loss_out.at[pl.ds(base, _C)])


@functools.lru_cache(maxsize=None)
def _p3():
    return pl.kernel(
        _p3_body,
        out_type=jax.ShapeDtypeStruct((_NPAD,), _f32),
        mesh=_mesh(),
        scratch_types=[
            pltpu.VMEM((_C,), _i32) for _ in range(4)
        ] + [pltpu.VMEM((_C,), _f32) for _ in range(11)] + [
            pltpu.SemaphoreType.DMA,
        ],
    )


def kernel(original_u, u_old, pos, node_type, extend_index, node_type_extended,
           original_block_metrics, extended_block_metrics, pos_extended,
           edge_node_xi_index, edge_node_eta_index, face_xi, face_eta, batch,
           pde_theta, relaxtion):
    del pde_theta
    py = _pad2d(pos[:, 1].astype(_f32))
    nt = _pad2d(node_type.astype(_i32))
    uold_f = _pad_flat(u_old[:, 0].astype(_f32))
    uori_f = _pad_flat(original_u[:, 0].astype(_f32))
    jo = _pad2d(original_block_metrics[:, 4].astype(_f32), fill=1)
    bat = _pad2d(batch.astype(_i32))
    xe = _pad2d(pos_extended[:, 0].astype(_f32))
    ye = _pad2d(pos_extended[:, 1].astype(_f32))
    g0 = _pad2d(extended_block_metrics[:, 0].astype(_f32))
    g1 = _pad2d(extended_block_metrics[:, 1].astype(_f32))
    g2 = _pad2d(extended_block_metrics[:, 2].astype(_f32))
    g3 = _pad2d(extended_block_metrics[:, 3].astype(_f32))
    gj = _pad2d(extended_block_metrics[:, 4].astype(_f32), fill=1)
    nte = _pad2d(node_type_extended.astype(_i32))
    relax = relaxtion.astype(_f32).reshape(1, _B)

    d_t, uvis, ubase, urel, uu, vv, mextf = _tc_prep(
        relax, py, nt, uold_f.reshape(_ROWS, 128), uori_f.reshape(_ROWS, 128),
        jo, bat, xe, ye, g0, g1, g2, g3, gj, nte)

    eidx = _pad_flat(extend_index.astype(_i32))
    uo_t, un_t = _p1()(uold_f, uori_f, d_t.reshape(-1), eidx,
                       mextf.reshape(-1))

    exl = _pad_flat(edge_node_xi_index[0].astype(_i32))
    exr = _pad_flat(edge_node_xi_index[1].astype(_i32))
    eed = _pad_flat(edge_node_eta_index[0].astype(_i32))
    eeu = _pad_flat(edge_node_eta_index[1].astype(_i32))
    fxo_t, fxn_t, feo_t, fen_t = _p2()(uo_t, un_t, uu.reshape(-1),
                                       vv.reshape(-1), exl, exr, eed, eeu)

    fxl = _pad_flat(face_xi[0].astype(_i32))
    fxr = _pad_flat(face_xi[1].astype(_i32))
    fed = _pad_flat(face_eta[0].astype(_i32))
    feu = _pad_flat(face_eta[1].astype(_i32))
    loss = _p3()(fxo_t, fxn_t, feo_t, fen_t, fxl, fxr, fed, feu,
                 ubase.reshape(-1), urel.reshape(-1))

    return (loss[:_N].reshape(_N, 1), uvis.reshape(-1)[:_N].reshape(_N, 1))


# speedup vs baseline: 4.3352x; 4.1291x over previous
import functools

import jax
import jax.numpy as jnp
from jax import lax
from jax.experimental import pallas as pl
from jax.experimental.pallas import tpu as pltpu
from jax.experimental.pallas import tpu_sc as plsc

_N = 50000
_B = 4
_VT_MAX = 0.385
_DT = 0.015625

_NW = 32
_SUB = 128
_JJ = 13
_C = _SUB * _JJ
_NPAD = _NW * _C
_ROWS = _NPAD // 128
_TB = 32
_KITERS = _C // 16
_SENT = 1e30

_f32 = jnp.float32
_i32 = jnp.int32


def _pad_flat(x, fill=0):
    x = x.reshape(-1)
    return jnp.pad(x, (0, _NPAD - x.shape[0]), constant_values=fill)


def _pad2d(x, fill=0):
    return _pad_flat(x, fill).reshape(_ROWS, 128)


def _prep_body(relax_ref, py, nt, uold, uori, jo, bat, xe, ye,
               g0, g1, g2, g3, gj, nte,
               d_o, uvis_o, ubase_o, urel_o, u_o, v_o, mextf_o):
    ubc = -jnp.tanh(py[...] * 0.5)
    mask_b = (nt[...] == 1) | (nt[...] == 2)
    d_o[...] = jnp.where(mask_b, ubc, jnp.float32(jnp.inf))
    uvis_o[...] = jnp.where(mask_b, ubc, uori[...])
    ubase_o[...] = (uori[...] - uold[...]) * (1.0 / _DT) / jo[...]
    b = bat[...]
    r0 = relax_ref[0, 0]
    r1 = relax_ref[0, 1]
    r2 = relax_ref[0, 2]
    r3 = relax_ref[0, 3]
    urel_o[...] = jnp.where(b == 0, r0,
                  jnp.where(b == 1, r1,
                  jnp.where(b == 2, r2, r3))).astype(_f32)
    x = xe[...]
    y = ye[...]
    r = jnp.sqrt(x * x + y * y)
    er = jnp.exp(-r)
    e2 = er * er
    sech = (2.0 * er) / (1.0 + e2)
    v_t = sech * sech * ((1.0 - e2) / (1.0 + e2))
    mask_r = r > 1e-12
    r_safe = jnp.where(mask_r, r, 1.0)
    omega = jnp.where(mask_r, (v_t / r_safe) / _VT_MAX, 0.0)
    a = -omega * y
    bb = omega * x
    u_o[...] = (a * g0[...] + bb * g1[...]) / gj[...]
    v_o[...] = (a * g2[...] + bb * g3[...]) / gj[...]
    mextf_o[...] = jnp.where((nte[...] == 1) | (nte[...] == 2), 1.0, 0.0)


def _tc_prep(relax, py, nt, uold, uori, jo, bat, xe, ye, g0, g1, g2, g3, gj,
             nte):
    blk = lambda: pl.BlockSpec((_TB, 128), lambda i: (i, 0))
    n_in = 14
    n_out = 7
    return pl.pallas_call(
        _prep_body,
        grid=(_ROWS // _TB,),
        in_specs=[pl.BlockSpec(memory_space=pltpu.SMEM)]
        + [blk() for _ in range(n_in)],
        out_specs=[blk() for _ in range(n_out)],
        out_shape=[jax.ShapeDtypeStruct((_ROWS, 128), _f32)
                   for _ in range(n_out)],
    )(relax, py, nt, uold, uori, jo, bat, xe, ye, g0, g1, g2, g3, gj, nte)


@functools.lru_cache(maxsize=None)
def _mesh():
    return plsc.VectorSubcoreMesh(core_axis_name="c", subcore_axis_name="s",
                                  num_cores=2, num_subcores=16)


def _wid():
    return lax.axis_index("s") * 2 + lax.axis_index("c")


_SS = _NPAD // 16


def _fire(tbl, idx_v, dst_v, sem):
    return [pltpu.async_copy(tbl.at[idx_v], dst_v, sem)]


def _stage(hbm, shared, sem):
    off = lax.axis_index("s") * _SS
    return [pltpu.async_copy(hbm.at[pl.ds(off, _SS)],
                             shared.at[pl.ds(off, _SS)], sem)]


def _p1_body(uold_t, uori_t, d_t, eidx, mextf,
             uo_out, un_out,
             idx_v, a_v, b_v, d_v, me_v, uo_v, un_v,
             uold_s, uori_s, d_s, sem):
    w = _wid()
    base = w * _C
    st = _stage(uold_t, uold_s, sem)
    st += _stage(uori_t, uori_s, sem)
    st += _stage(d_t, d_s, sem)
    pltpu.sync_copy(eidx.at[pl.ds(base, _C)], idx_v)
    pltpu.sync_copy(mextf.at[pl.ds(base, _C)], me_v)
    for d in st:
        d.wait()
    plsc.subcore_barrier()
    descs = _fire(uold_s, idx_v, a_v, sem)
    descs += _fire(uori_s, idx_v, b_v, sem)
    descs += _fire(d_s, idx_v, d_v, sem)
    for d in descs:
        d.wait()

    def body(k, carry):
        s = pl.ds(k * 16, 16)
        dg = d_v[s]
        sel = (me_v[s] > 0.5) & (dg < _SENT)
        uo_v[s] = jnp.where(sel, dg, a_v[s])
        un_v[s] = jnp.where(sel, dg, b_v[s])
        return carry

    lax.fori_loop(0, _KITERS, body, 0)
    pltpu.sync_copy(uo_v, uo_out.at[pl.ds(base, _C)])
    pltpu.sync_copy(un_v, un_out.at[pl.ds(base, _C)])


@functools.lru_cache(maxsize=None)
def _p1():
    return pl.kernel(
        _p1_body,
        out_type=[jax.ShapeDtypeStruct((_NPAD,), _f32),
                  jax.ShapeDtypeStruct((_NPAD,), _f32)],
        mesh=_mesh(),
        scratch_types=[
            pltpu.VMEM((_C,), _i32),
            pltpu.VMEM((_C,), _f32),
            pltpu.VMEM((_C,), _f32),
            pltpu.VMEM((_C,), _f32),
            pltpu.VMEM((_C,), _f32),
            pltpu.VMEM((_C,), _f32),
            pltpu.VMEM((_C,), _f32),
            pltpu.VMEM_SHARED((_NPAD,), _f32),
            pltpu.VMEM_SHARED((_NPAD,), _f32),
            pltpu.VMEM_SHARED((_NPAD,), _f32),
            pltpu.SemaphoreType.DMA,
        ],
    )


def _p2_body(uo_t, un_t, uu_t, vv_t, exl, exr, eed, eeu,
             fxo_out, fxn_out, feo_out, fen_out,
             il_v, ir_v, uol_v, uor_v, unl_v, unr_v, cl_v, cr_v,
             fo_v, fn_v, uo_s, un_s, uu_s, vv_s, sem):
    w = _wid()
    base = w * _C
    st = _stage(uo_t, uo_s, sem)
    st += _stage(un_t, un_s, sem)
    st += _stage(uu_t, uu_s, sem)
    st += _stage(vv_t, vv_s, sem)
    for d in st:
        d.wait()
    plsc.subcore_barrier()
    for il, ir, ct, oo, on in ((exl, exr, uu_s, fxo_out, fxn_out),
                               (eed, eeu, vv_s, feo_out, fen_out)):
        pltpu.sync_copy(il.at[pl.ds(base, _C)], il_v)
        pltpu.sync_copy(ir.at[pl.ds(base, _C)], ir_v)
        descs = _fire(uo_s, il_v, uol_v, sem)
        descs += _fire(uo_s, ir_v, uor_v, sem)
        descs += _fire(un_s, il_v, unl_v, sem)
        descs += _fire(un_s, ir_v, unr_v, sem)
        descs += _fire(ct, il_v, cl_v, sem)
        descs += _fire(ct, ir_v, cr_v, sem)
        for d in descs:
            d.wait()

        def body(k, carry):
            s = pl.ds(k * 16, 16)
            cc = 0.25 * (cl_v[s] + cr_v[s])
            fo_v[s] = (uol_v[s] + uor_v[s]) * cc
            fn_v[s] = (unl_v[s] + unr_v[s]) * cc
            return carry

        lax.fori_loop(0, _KITERS, body, 0)
        pltpu.sync_copy(fo_v, oo.at[pl.ds(base, _C)])
        pltpu.sync_copy(fn_v, on.at[pl.ds(base, _C)])


@functools.lru_cache(maxsize=None)
def _p2():
    return pl.kernel(
        _p2_body,
        out_type=[jax.ShapeDtypeStruct((_NPAD,), _f32) for _ in range(4)],
        mesh=_mesh(),
        scratch_types=[
            pltpu.VMEM((_C,), _i32),
            pltpu.VMEM((_C,), _i32),
        ] + [pltpu.VMEM((_C,), _f32) for _ in range(8)] + [
            pltpu.VMEM_SHARED((_NPAD,), _f32) for _ in range(4)
        ] + [
            pltpu.SemaphoreType.DMA,
        ],
    )


def _p3_body(fxo_t, fxn_t, feo_t, fen_t, fxl, fxr, fed, feu, ubase, urel,
             loss_out,
             ia, ib, ic, idd, xol, xor_, xnl, xnr, eod, eou, end_, enu,
             ub_v, ur_v, loss_v, fxo_s, fxn_s, feo_s, fen_s, sem):
    w = _wid()
    base = w * _C
    st = _stage(fxo_t, fxo_s, sem)
    st += _stage(fxn_t, fxn_s, sem)
    st += _stage(feo_t, feo_s, sem)
    st += _stage(fen_t, fen_s, sem)
    pltpu.sync_copy(fxl.at[pl.ds(base, _C)], ia)
    pltpu.sync_copy(fxr.at[pl.ds(base, _C)], ib)
    pltpu.sync_copy(fed.at[pl.ds(base, _C)], ic)
    pltpu.sync_copy(feu.at[pl.ds(base, _C)], idd)
    for d in st:
        d.wait()
    plsc.subcore_barrier()
    descs = _fire(fxo_s, ia, xol, sem)
    descs += _fire(fxo_s, ib, xor_, sem)
    descs += _fire(fxn_s, ia, xnl, sem)
    descs += _fire(fxn_s, ib, xnr, sem)
    descs += _fire(feo_s, ic, eod, sem)
    descs += _fire(feo_s, idd, eou, sem)
    descs += _fire(fen_s, ic, end_, sem)
    descs += _fire(fen_s, idd, enu, sem)
    pltpu.sync_copy(ubase.at[pl.ds(base, _C)], ub_v)
    pltpu.sync_copy(urel.at[pl.ds(base, _C)], ur_v)
    for d in descs:
        d.wait()

    def body(k, carry):
        s = pl.ds(k * 16, 16)
        co = (xor_[s] - xol[s]) + (eou[s] - eod[s])
        cn = (xnr[s] - xnl[s]) + (enu[s] - end_[s])
        ur = ur_v[s]
        loss_v[s] = ub_v[s] + ur * co + (1.0 - ur) * cn
        return carry

    lax.fori_loop(0, _KITERS, body, 0)
    pltpu.sync_copy(loss_v, loss_out.at[pl.ds(base, _C)])


@functools.lru_cache(maxsize=None)
def _p3():
    return pl.kernel(
        _p3_body,
        out_type=jax.ShapeDtypeStruct((_NPAD,), _f32),
        mesh=_mesh(),
        scratch_types=[
            pltpu.VMEM((_C,), _i32) for _ in range(4)
        ] + [pltpu.VMEM((_C,), _f32) for _ in range(11)] + [
            pltpu.VMEM_SHARED((_NPAD,), _f32) for _ in range(4)
        ] + [
            pltpu.SemaphoreType.DMA,
        ],
    )


def kernel(original_u, u_old, pos, node_type, extend_index, node_type_extended,
           original_block_metrics, extended_block_metrics, pos_extended,
           edge_node_xi_index, edge_node_eta_index, face_xi, face_eta, batch,
           pde_theta, relaxtion):
    del pde_theta
    py = _pad2d(pos[:, 1].astype(_f32))
    nt = _pad2d(node_type.astype(_i32))
    uold_f = _pad_flat(u_old[:, 0].astype(_f32))
    uori_f = _pad_flat(original_u[:, 0].astype(_f32))
    jo = _pad2d(original_block_metrics[:, 4].astype(_f32), fill=1)
    bat = _pad2d(batch.astype(_i32))
    xe = _pad2d(pos_extended[:, 0].astype(_f32))
    ye = _pad2d(pos_extended[:, 1].astype(_f32))
    g0 = _pad2d(extended_block_metrics[:, 0].astype(_f32))
    g1 = _pad2d(extended_block_metrics[:, 1].astype(_f32))
    g2 = _pad2d(extended_block_metrics[:, 2].astype(_f32))
    g3 = _pad2d(extended_block_metrics[:, 3].astype(_f32))
    gj = _pad2d(extended_block_metrics[:, 4].astype(_f32), fill=1)
    nte = _pad2d(node_type_extended.astype(_i32))
    relax = relaxtion.astype(_f32).reshape(1, _B)

    d_t, uvis, ubase, urel, uu, vv, mextf = _tc_prep(
        relax, py, nt, uold_f.reshape(_ROWS, 128), uori_f.reshape(_ROWS, 128),
        jo, bat, xe, ye, g0, g1, g2, g3, gj, nte)

    eidx = _pad_flat(extend_index.astype(_i32))
    uo_t, un_t = _p1()(uold_f, uori_f, d_t.reshape(-1), eidx,
                       mextf.reshape(-1))

    exl = _pad_flat(edge_node_xi_index[0].astype(_i32))
    exr = _pad_flat(edge_node_xi_index[1].astype(_i32))
    eed = _pad_flat(edge_node_eta_index[0].astype(_i32))
    eeu = _pad_flat(edge_node_eta_index[1].astype(_i32))
    fxo_t, fxn_t, feo_t, fen_t = _p2()(uo_t, un_t, uu.reshape(-1),
                                       vv.reshape(-1), exl, exr, eed, eeu)

    fxl = _pad_flat(face_xi[0].astype(_i32))
    fxr = _pad_flat(face_xi[1].astype(_i32))
    fed = _pad_flat(face_eta[0].astype(_i32))
    feu = _pad_flat(face_eta[1].astype(_i32))
    loss = _p3()(fxo_t, fxn_t, feo_t, fen_t, fxl, fxr, fed, feu,
                 ubase.reshape(-1), urel.reshape(-1))

    return (loss[:_N].reshape(_N, 1), uvis.reshape(-1)[:_N].reshape(_N, 1))
